# baseline (speedup 1.0000x reference)
import functools

import jax
import jax.numpy as jnp
from jax import lax
from jax.experimental import pallas as pl
from jax.experimental.pallas import tpu as pltpu
from jax.experimental.pallas import tpu_sc as plsc

_TRUNC = 32
_NEG_SENTINEL = -1.0

_NC = 2
_NS = 16
_LANES = 16
_NW = _NC * _NS

_N = 8192
_R = 4608
_ROWS_PT = (_N - _R) // _NW
_NBANDS = _ROWS_PT // 8
_BLK = 512


def _topk_kernel(x_ref, vals_ref, idx_ref, z_ref):
    x = x_ref[...]
    b, n = x.shape
    col = lax.broadcasted_iota(jnp.int32, (b, n), 1)
    kcol = lax.broadcasted_iota(jnp.int32, (b, _TRUNC), 1)

    def body(k, carry):
        xm, zlog, sel, vacc, iacc = carry
        rowmax = jnp.max(xm, axis=1, keepdims=True)
        logv = jnp.clip(jnp.log(rowmax), -1000.0, None) + 50.0
        poscand = jnp.where(xm == rowmax, col, n)
        firstpos = jnp.min(poscand, axis=1, keepdims=True)
        mask = col == firstpos
        ksel = kcol == k
        vacc = jnp.where(ksel, logv, vacc)
        iacc = jnp.where(ksel, firstpos, iacc)
        zlog = zlog + jnp.where(mask, logv, 0.0)
        sel = sel + jnp.where(mask, 1.0, 0.0)
        xm = jnp.where(mask, _NEG_SENTINEL, xm)
        return xm, zlog, sel, vacc, iacc

    zeros = jnp.zeros((b, n), jnp.float32)
    vacc0 = jnp.zeros((b, _TRUNC), jnp.float32)
    iacc0 = jnp.zeros((b, _TRUNC), jnp.int32)
    _, zlog, sel, vacc, iacc = lax.fori_loop(
        0, _TRUNC, body, (x, zeros, zeros, vacc0, iacc0))
    shift = jax.nn.relu(-jnp.min(vacc, axis=1, keepdims=True))
    v = vacc + shift
    norm = jnp.clip(jnp.sqrt(jnp.sum(v * v, axis=1, keepdims=True)),
                    1e-12, None)
    vals_ref[...] = v / norm
    idx_ref[...] = iacc
    z_ref[...] = sel * (zlog + shift) / norm


_HW = _N // 2


def _sc_body(idx_hbm, val_hbm, w1_hbm, out_hbm, idx_v, val_v, jl_v, vl_v,
             buf_a, buf_b, hacc, sem_a, sem_b):
    cid = lax.axis_index("c")
    sid = lax.axis_index("s")
    wid = sid * _NC + cid
    row0 = _R + wid * _ROWS_PT

    pltpu.sync_copy(idx_hbm, idx_v)
    pltpu.sync_copy(val_hbm, val_v)
    lane = lax.iota(jnp.int32, _LANES)

    for half in range(2):
        c0 = half * _HW
        for ch in range(_TRUNC * 8 // _LANES):
            jv = idx_v[pl.ds(ch * _LANES, _LANES)]
            vv = val_v[pl.ds(ch * _LANES, _LANES)]
            inr = (jv >= c0) & (jv < c0 + _HW)
            jl = jnp.clip(jv - c0, 0, _HW - 1)
            off = half * (_TRUNC * 8) + ch * _LANES
            jl_v[pl.ds(off, _LANES)] = jl
            vl_v[pl.ds(off, _LANES)] = jnp.where(inr, vv, 0.0)

    def _src(bi, half):
        return w1_hbm.at[pl.ds(row0 + bi * 8, 8), pl.ds(half * _HW, _HW)]

    def _phase(bi, half, buf, first):
        def rowloop(r, carry2):
            iv = jnp.full((_LANES,), r, jnp.int32)
            sums = []
            for b in range(8):
                parts = None
                for c in range(2):
                    off = half * (_TRUNC * 8) + b * _TRUNC + c * 16
                    jv = jl_v[pl.ds(off, 16)]
                    vv = vl_v[pl.ds(off, 16)]
                    gv = plsc.load_gather(buf, [iv, jv]) * vv
                    parts = gv if parts is None else parts + gv
                sums.append(plsc.cumsum(parts)[15])
            hv = jnp.zeros((_LANES,), jnp.float32)
            for b in range(8):
                hv = jnp.where(lane == b, sums[b], hv)
            ridx = jnp.full((_LANES,), bi * 8 + r, jnp.int32)
            if first:
                plsc.store_scatter(hacc, [ridx, lane], hv, mask=lane < 8)
            else:
                plsc.addupdate_scatter(hacc, [ridx, lane], hv, mask=lane < 8)
            return carry2

        lax.fori_loop(0, 8, rowloop, 0)

    pltpu.async_copy(_src(0, 0), buf_a, sem_a)

    def bandloop(bi, carry):
        pltpu.async_copy(_src(bi, 1), buf_b, sem_b)
        pltpu.make_async_copy(_src(bi, 0), buf_a, sem_a).wait()
        _phase(bi, 0, buf_a, first=True)

        @pl.when(bi + 1 < _NBANDS)
        def _():
            pltpu.async_copy(_src(bi + 1, 0), buf_a, sem_a)

        pltpu.make_async_copy(_src(bi, 1), buf_b, sem_b).wait()
        _phase(bi, 1, buf_b, first=False)
        return carry

    lax.fori_loop(0, _NBANDS, bandloop, 0)
    pltpu.sync_copy(hacc, out_hbm.at[pl.ds(wid * _ROWS_PT, _ROWS_PT),
                                     pl.ds(0, 8)])


@functools.cache
def _sc_gather_mlp():
    return pl.kernel(
        _sc_body,
        out_type=jax.ShapeDtypeStruct((_N - _R, 8), jnp.float32),
        mesh=plsc.VectorSubcoreMesh(
            core_axis_name="c", subcore_axis_name="s",
            num_cores=_NC, num_subcores=_NS),
        scratch_types=[
            pltpu.VMEM((_TRUNC * 8,), jnp.int32),
            pltpu.VMEM((_TRUNC * 8,), jnp.float32),
            pltpu.VMEM((2 * _TRUNC * 8,), jnp.int32),
            pltpu.VMEM((2 * _TRUNC * 8,), jnp.float32),
            pltpu.VMEM((8, _HW), jnp.float32),
            pltpu.VMEM((8, _HW), jnp.float32),
            pltpu.VMEM((_ROWS_PT, 8), jnp.float32),
            pltpu.SemaphoreType.DMA,
            pltpu.SemaphoreType.DMA,
        ],
        compiler_params=pltpu.CompilerParams(
            use_tc_tiling_on_sc=True, needs_layout_passes=False),
    )


def _mlp_part1(z_ref, w1_ref, b1_ref, w2_ref, out_ref, acc_ref):
    j = pl.program_id(0)

    @pl.when(j == 0)
    def _():
        acc_ref[...] = jnp.zeros_like(acc_ref)

    h = lax.dot_general(
        z_ref[...], w1_ref[...], (((1,), (1,)), ((), ())),
        preferred_element_type=jnp.float32) + b1_ref[...]
    h = jnp.where(h >= 0, h, 0.2 * h)
    acc_ref[...] += lax.dot_general(
        h, w2_ref[...], (((1,), (1,)), ((), ())),
        preferred_element_type=jnp.float32)

    @pl.when(j == pl.num_programs(0) - 1)
    def _():
        out_ref[...] = acc_ref[...]


def _mlp_part2(ht_ref, b1c_ref, w2_ref, part_ref, b2_ref, out_ref, acc_ref):
    j = pl.program_id(0)

    @pl.when(j == 0)
    def _():
        acc_ref[...] = jnp.zeros_like(acc_ref)

    h = ht_ref[...] + b1c_ref[pl.ds(j * _BLK, _BLK), :]
    h = jnp.where(h >= 0, h, 0.2 * h)
    acc_ref[...] += lax.dot_general(
        h, w2_ref[...], (((0,), (1,)), ((), ())),
        preferred_element_type=jnp.float32)

    @pl.when(j == pl.num_programs(0) - 1)
    def _():
        out_ref[...] = acc_ref[...] + part_ref[...] + b2_ref[...]


@jax.jit
def _impl(x, W1, b1, W2, b2):
    batch, n = x.shape
    out_dim = W2.shape[0]

    vals, idx, z = pl.pallas_call(
        _topk_kernel,
        out_shape=(
            jax.ShapeDtypeStruct((batch, _TRUNC), jnp.float32),
            jax.ShapeDtypeStruct((batch, _TRUNC), jnp.int32),
            jax.ShapeDtypeStruct((batch, n), jnp.float32),
        ),
    )(x)

    h_sc_t = _sc_gather_mlp()(idx.reshape(-1), vals.reshape(-1), W1)

    b1r = b1.reshape(1, -1)
    part = pl.pallas_call(
        _mlp_part1,
        grid=(_R // _BLK,),
        in_specs=[
            pl.BlockSpec((batch, n), lambda j: (0, 0)),
            pl.BlockSpec((_BLK, n), lambda j: (j, 0)),
            pl.BlockSpec((1, _BLK), lambda j: (0, j)),
            pl.BlockSpec((out_dim, _BLK), lambda j: (0, j)),
        ],
        out_specs=pl.BlockSpec((batch, out_dim), lambda j: (0, 0)),
        out_shape=jax.ShapeDtypeStruct((batch, out_dim), jnp.float32),
        scratch_shapes=[pltpu.VMEM((batch, out_dim), jnp.float32)],
    )(z, W1, b1r, W2)

    nblk2 = (n - _R) // _BLK
    b1col = b1[_R:].reshape(-1, 1)
    out = pl.pallas_call(
        _mlp_part2,
        grid=(nblk2,),
        in_specs=[
            pl.BlockSpec((_BLK, batch), lambda j: (j, 0)),
            pl.BlockSpec((n - _R, 1), lambda j: (0, 0)),
            pl.BlockSpec((out_dim, _BLK), lambda j: (0, (_R // _BLK) + j)),
            pl.BlockSpec((batch, out_dim), lambda j: (0, 0)),
            pl.BlockSpec((1, out_dim), lambda j: (0, 0)),
        ],
        out_specs=pl.BlockSpec((batch, out_dim), lambda j: (0, 0)),
        out_shape=jax.ShapeDtypeStruct((batch, out_dim), jnp.float32),
        scratch_shapes=[pltpu.VMEM((batch, out_dim), jnp.float32)],
    )(h_sc_t, b1col, W2, part, b2.reshape(1, -1))
    return out


def kernel(x, W1, b1, W2, b2):
    return _impl(x, W1, b1, W2, b2)


# speedup vs baseline: 1.4008x; 1.4008x over previous
import functools

import jax
import jax.numpy as jnp
from jax import lax
from jax.experimental import pallas as pl
from jax.experimental.pallas import tpu as pltpu

_TRUNC = 32
_N = 8192
_BLK = 512
_BISECT_ITERS = 34


def _build_z(x):
    b, n = x.shape
    lo = jnp.zeros((b, 1), jnp.float32)
    hi = jnp.ones((b, 1), jnp.float32)
    kf = jnp.float32(_TRUNC)

    def bis(_, carry):
        lo, hi = carry
        mid = 0.5 * (lo + hi)
        cnt = jnp.sum(jnp.where(x > mid, 1.0, 0.0), axis=1, keepdims=True)
        ge = cnt >= kf
        return jnp.where(ge, mid, lo), jnp.where(ge, hi, mid)

    lo, hi = lax.fori_loop(0, _BISECT_ITERS, bis, (lo, hi))
    v32 = jnp.min(jnp.where(x > lo, x, 2.0), axis=1, keepdims=True)
    gt = x > v32
    cgt = jnp.sum(jnp.where(gt, 1.0, 0.0), axis=1, keepdims=True)
    need = kf - cgt
    tie = x == v32
    pre = jnp.where(tie, 1.0, 0.0)
    d = 1
    while d < n:
        pre = pre + jnp.concatenate(
            [jnp.zeros((b, d), jnp.float32), pre[:, :-d]], axis=1)
        d *= 2
    sel = gt | (tie & (pre <= need))
    logv = jnp.clip(jnp.log(x), -1000.0, None) + 50.0
    minlog = jnp.clip(jnp.log(v32), -1000.0, None) + 50.0
    shift = jax.nn.relu(-minlog)
    z = jnp.where(sel, logv + shift, 0.0)
    norm = jnp.sqrt(jnp.sum(z * z, axis=1, keepdims=True))
    return z / jnp.clip(norm, 1e-12, None)


def _fused_kernel(x_ref, w1_ref, b1_ref, w2_ref, b2_ref, out_ref,
                  z_ref, acc_ref):
    j = pl.program_id(0)

    @pl.when(j == 0)
    def _():
        z_ref[...] = _build_z(x_ref[...])
        acc_ref[...] = jnp.zeros_like(acc_ref)

    h = lax.dot_general(
        z_ref[...], w1_ref[...], (((1,), (1,)), ((), ())),
        preferred_element_type=jnp.float32) + b1_ref[...]
    h = jnp.where(h >= 0, h, 0.2 * h)
    acc_ref[...] += lax.dot_general(
        h, w2_ref[...], (((1,), (1,)), ((), ())),
        preferred_element_type=jnp.float32)

    @pl.when(j == pl.num_programs(0) - 1)
    def _():
        out_ref[...] = acc_ref[...] + b2_ref[...]


@functools.partial(jax.jit, static_argnames=("interpret",))
def _impl(x, W1, b1, W2, b2, interpret=False):
    batch, n = x.shape
    out_dim = W2.shape[0]
    return pl.pallas_call(
        _fused_kernel,
        grid=(n // _BLK,),
        in_specs=[
            pl.BlockSpec((batch, n), lambda j: (0, 0)),
            pl.BlockSpec((_BLK, n), lambda j: (j, 0)),
            pl.BlockSpec((1, _BLK), lambda j: (0, j)),
            pl.BlockSpec((out_dim, _BLK), lambda j: (0, j)),
            pl.BlockSpec((1, out_dim), lambda j: (0, 0)),
        ],
        out_specs=pl.BlockSpec((batch, out_dim), lambda j: (0, 0)),
        out_shape=jax.ShapeDtypeStruct((batch, out_dim), jnp.float32),
        scratch_shapes=[
            pltpu.VMEM((batch, n), jnp.float32),
            pltpu.VMEM((batch, out_dim), jnp.float32),
        ],
        interpret=interpret,
    )(x, W1, b1.reshape(1, -1), W2, b2.reshape(1, -1))


def kernel(x, W1, b1, W2, b2):
    return _impl(x, W1, b1, W2, b2)


# speedup vs baseline: 1.4204x; 1.0140x over previous
import functools

import jax
import jax.numpy as jnp
from jax import lax
from jax.experimental import pallas as pl
from jax.experimental.pallas import tpu as pltpu

_TRUNC = 32
_N = 8192
_BLK = 512
_BISECT_ITERS = 34


def _build_z(x):
    b, n = x.shape
    lo = jnp.zeros((b, 1), jnp.float32)
    hi = jnp.ones((b, 1), jnp.float32)
    kf = jnp.float32(_TRUNC)

    def bis(_, carry):
        lo, hi = carry
        mid = 0.5 * (lo + hi)
        cnt = jnp.sum(jnp.where(x > mid, 1.0, 0.0), axis=1, keepdims=True)
        ge = cnt >= kf
        return jnp.where(ge, mid, lo), jnp.where(ge, hi, mid)

    lo, hi = lax.fori_loop(0, _BISECT_ITERS, bis, (lo, hi))
    v32 = jnp.min(jnp.where(x > lo, x, 2.0), axis=1, keepdims=True)
    gt = x > v32
    cgt = jnp.sum(jnp.where(gt, 1.0, 0.0), axis=1, keepdims=True)
    need = kf - cgt
    tie = x == v32
    pre = jnp.where(tie, 1.0, 0.0)
    d = 1
    while d < n:
        pre = pre + jnp.concatenate(
            [jnp.zeros((b, d), jnp.float32), pre[:, :-d]], axis=1)
        d *= 2
    sel = gt | (tie & (pre <= need))
    logv = jnp.clip(jnp.log(x), -1000.0, None) + 50.0
    minlog = jnp.clip(jnp.log(v32), -1000.0, None) + 50.0
    shift = jax.nn.relu(-minlog)
    z = jnp.where(sel, logv + shift, 0.0)
    norm = jnp.sqrt(jnp.sum(z * z, axis=1, keepdims=True))
    return z / jnp.clip(norm, 1e-12, None)


def _fused_kernel(x_ref, w1_ref, b1_ref, w2_ref, b2_ref, out_ref,
                  z_ref, acc_ref):
    j = pl.program_id(0)

    @pl.when(j == 0)
    def _():
        z_ref[...] = _build_z(x_ref[...])
        acc_ref[...] = jnp.zeros_like(acc_ref)

    h = lax.dot_general(
        z_ref[...], w1_ref[...], (((1,), (1,)), ((), ())),
        preferred_element_type=jnp.float32) + b1_ref[...][None, :]
    h = jnp.where(h >= 0, h, 0.2 * h)
    acc_ref[...] += lax.dot_general(
        h, w2_ref[...], (((1,), (1,)), ((), ())),
        preferred_element_type=jnp.float32)

    @pl.when(j == pl.num_programs(0) - 1)
    def _():
        out_ref[...] = acc_ref[...] + b2_ref[...][None, :]


@functools.partial(jax.jit, static_argnames=("interpret",))
def _impl(x, W1, b1, W2, b2, interpret=False):
    batch, n = x.shape
    out_dim = W2.shape[0]
    return pl.pallas_call(
        _fused_kernel,
        grid=(n // _BLK,),
        in_specs=[
            pl.BlockSpec((batch, n), lambda j: (0, 0)),
            pl.BlockSpec((_BLK, n), lambda j: (j, 0)),
            pl.BlockSpec((_BLK,), lambda j: (j,)),
            pl.BlockSpec((out_dim, _BLK), lambda j: (0, j)),
            pl.BlockSpec((out_dim,), lambda j: (0,)),
        ],
        out_specs=pl.BlockSpec((batch, out_dim), lambda j: (0, 0)),
        out_shape=jax.ShapeDtypeStruct((batch, out_dim), jnp.float32),
        scratch_shapes=[
            pltpu.VMEM((batch, n), jnp.float32),
            pltpu.VMEM((batch, out_dim), jnp.float32),
        ],
        interpret=interpret,
    )(x, W1, b1, W2, b2)


def kernel(x, W1, b1, W2, b2):
    return _impl(x, W1, b1, W2, b2)


# speedup vs baseline: 1.4456x; 1.0178x over previous
import functools

import jax
import jax.numpy as jnp
from jax import lax
from jax.experimental import pallas as pl
from jax.experimental.pallas import tpu as pltpu

_TRUNC = 32
_N = 8192
_BLK = 256
_BISECT_ITERS = 34


def _build_z(x):
    b, n = x.shape
    lo = jnp.zeros((b, 1), jnp.float32)
    hi = jnp.ones((b, 1), jnp.float32)
    kf = jnp.float32(_TRUNC)

    def bis(_, carry):
        lo, hi = carry
        mid = 0.5 * (lo + hi)
        cnt = jnp.sum(jnp.where(x > mid, 1.0, 0.0), axis=1, keepdims=True)
        ge = cnt >= kf
        return jnp.where(ge, mid, lo), jnp.where(ge, hi, mid)

    lo, hi = lax.fori_loop(0, _BISECT_ITERS, bis, (lo, hi))
    v32 = jnp.min(jnp.where(x > lo, x, 2.0), axis=1, keepdims=True)
    gt = x > v32
    cgt = jnp.sum(jnp.where(gt, 1.0, 0.0), axis=1, keepdims=True)
    need = kf - cgt
    tie = x == v32
    pre = jnp.where(tie, 1.0, 0.0)
    d = 1
    while d < n:
        pre = pre + jnp.concatenate(
            [jnp.zeros((b, d), jnp.float32), pre[:, :-d]], axis=1)
        d *= 2
    sel = gt | (tie & (pre <= need))
    logv = jnp.clip(jnp.log(x), -1000.0, None) + 50.0
    minlog = jnp.clip(jnp.log(v32), -1000.0, None) + 50.0
    shift = jax.nn.relu(-minlog)
    z = jnp.where(sel, logv + shift, 0.0)
    norm = jnp.sqrt(jnp.sum(z * z, axis=1, keepdims=True))
    return z / jnp.clip(norm, 1e-12, None)


def _fused_kernel(x_ref, w1a_ref, w1b_ref, b1a_ref, b1b_ref,
                  w2a_ref, w2b_ref, b2_ref, out_ref, z_ref, acc_ref):
    j = pl.program_id(0)

    @pl.when(j == 0)
    def _():
        z_ref[...] = _build_z(x_ref[...])
        acc_ref[...] = jnp.zeros_like(acc_ref)

    acc = acc_ref[...]
    for w1_ref, b1_ref, w2_ref in ((w1a_ref, b1a_ref, w2a_ref),
                                   (w1b_ref, b1b_ref, w2b_ref)):
        h = lax.dot_general(
            z_ref[...], w1_ref[...], (((1,), (1,)), ((), ())),
            preferred_element_type=jnp.float32) + b1_ref[...][None, :]
        h = jnp.where(h >= 0, h, 0.2 * h)
        acc = acc + lax.dot_general(
            h, w2_ref[...], (((1,), (1,)), ((), ())),
            preferred_element_type=jnp.float32)
    acc_ref[...] = acc

    @pl.when(j == pl.num_programs(0) - 1)
    def _():
        out_ref[...] = acc_ref[...] + b2_ref[...][None, :]


@functools.partial(jax.jit, static_argnames=("interpret",))
def _impl(x, W1, b1, W2, b2, interpret=False):
    batch, n = x.shape
    out_dim = W2.shape[0]
    half_steps = n // _BLK // 2
    return pl.pallas_call(
        _fused_kernel,
        grid=(half_steps,),
        in_specs=[
            pl.BlockSpec((batch, n), lambda j: (0, 0)),
            pl.BlockSpec((_BLK, n), lambda j: (j, 0)),
            pl.BlockSpec((_BLK, n), lambda j: (j + half_steps, 0)),
            pl.BlockSpec((_BLK,), lambda j: (j,)),
            pl.BlockSpec((_BLK,), lambda j: (j + half_steps,)),
            pl.BlockSpec((out_dim, _BLK), lambda j: (0, j)),
            pl.BlockSpec((out_dim, _BLK), lambda j: (0, j + half_steps)),
            pl.BlockSpec((out_dim,), lambda j: (0,)),
        ],
        out_specs=pl.BlockSpec((batch, out_dim), lambda j: (0, 0)),
        out_shape=jax.ShapeDtypeStruct((batch, out_dim), jnp.float32),
        scratch_shapes=[
            pltpu.VMEM((batch, n), jnp.float32),
            pltpu.VMEM((batch, out_dim), jnp.float32),
        ],
        interpret=interpret,
    )(x, W1, W1, b1, b1, W2, W2, b2)


def kernel(x, W1, b1, W2, b2):
    return _impl(x, W1, b1, W2, b2)
